# baseline (speedup 1.0000x reference)
import functools

import jax
import jax.numpy as jnp
from jax import lax
from jax.experimental import pallas as pl
from jax.experimental.pallas import tpu as pltpu
from jax.experimental.pallas import tpu_sc as plsc


def _gather_kernel(n_tokens, dim, n_workers, chunk):
    n_chunks = n_tokens // (n_workers * chunk)
    mesh = plsc.VectorSubcoreMesh(core_axis_name="c", subcore_axis_name="s")

    @functools.partial(
        pl.kernel,
        mesh=mesh,
        out_type=jax.ShapeDtypeStruct((n_tokens, dim), jnp.float32),
        scratch_types=[
            pltpu.VMEM((chunk,), jnp.int32),
            pltpu.VMEM((chunk, dim), jnp.float32),
            pltpu.SemaphoreType.DMA,
        ],
        compiler_params=pltpu.CompilerParams(use_tc_tiling_on_sc=False),
    )
    def k(idx_hbm, table_hbm, out_hbm, idx_v, rows_v, sem):
        wid = lax.axis_index("s") * 2 + lax.axis_index("c")
        base = wid * (n_chunks * chunk)

        def body(i, carry):
            off = base + i * chunk
            pltpu.sync_copy(idx_hbm.at[pl.ds(off, chunk)], idx_v)
            pltpu.async_copy(table_hbm.at[idx_v], rows_v, sem).wait()
            pltpu.sync_copy(rows_v, out_hbm.at[pl.ds(off, chunk)])
            return carry

        lax.fori_loop(0, n_chunks, body, 0)

    return k


def kernel(token_ids, weight):
    b, s = token_ids.shape
    v, d = weight.shape
    n = b * s
    n_workers = 32
    chunk = 1024
    assert n % (n_workers * chunk) == 0
    flat = token_ids.reshape(n).astype(jnp.int32)
    out = _gather_kernel(n, d, n_workers, chunk)(flat, weight)
    return out.reshape(b, s, d)


# speedup vs baseline: 1.0100x; 1.0100x over previous
import functools

import jax
import jax.numpy as jnp
from jax import lax
from jax.experimental import pallas as pl
from jax.experimental.pallas import tpu as pltpu
from jax.experimental.pallas import tpu_sc as plsc

_N_WORKERS = 32


def _gather_kernel(n_tokens, dim, chunk, nbuf, dist):
    per_w = n_tokens // _N_WORKERS
    n_chunks = per_w // chunk
    mesh = plsc.VectorSubcoreMesh(core_axis_name="c", subcore_axis_name="s")

    @functools.partial(
        pl.kernel,
        mesh=mesh,
        out_type=jax.ShapeDtypeStruct((n_tokens, dim), jnp.float32),
        scratch_types=(
            [
                pltpu.VMEM((n_chunks, chunk), jnp.int32),
                pltpu.VMEM((nbuf, chunk, dim), jnp.float32),
            ]
            + [pltpu.SemaphoreType.DMA] * (2 * nbuf)
        ),
        compiler_params=pltpu.CompilerParams(use_tc_tiling_on_sc=False),
    )
    def k(idx_hbm, table_hbm, out_hbm, idx_v, rows_v, *sems):
        gsems = sems[:nbuf]
        wsems = sems[nbuf:]
        wid = lax.axis_index("s") * 2 + lax.axis_index("c")
        row0 = wid * n_chunks
        base = wid * per_w

        pltpu.sync_copy(idx_hbm.at[pl.ds(row0, n_chunks)], idx_v)

        pending_g = {}
        pending_w = {}

        def start_gather(j):
            b = j % nbuf
            pending_g[b] = pltpu.async_copy(
                table_hbm.at[idx_v.at[j]], rows_v.at[b], gsems[b]
            )

        for j in range(min(dist, n_chunks)):
            start_gather(j)
        for i in range(n_chunks):
            b = i % nbuf
            pending_g.pop(b).wait()
            pending_w[b] = pltpu.async_copy(
                rows_v.at[b], out_hbm.at[pl.ds(base + i * chunk, chunk)], wsems[b]
            )
            j = i + dist
            if j < n_chunks:
                bj = j % nbuf
                if bj in pending_w:
                    pending_w.pop(bj).wait()
                start_gather(j)
        for w in pending_w.values():
            w.wait()

    return k


def kernel(token_ids, weight):
    b, s = token_ids.shape
    v, d = weight.shape
    n = b * s
    chunk = 512
    per_w = n // _N_WORKERS
    assert per_w % chunk == 0
    flat = token_ids.reshape(n // chunk, chunk).astype(jnp.int32)
    out = _gather_kernel(n, d, chunk, nbuf=3, dist=2)(flat, weight)
    return out.reshape(b, s, d)


# speedup vs baseline: 1.0112x; 1.0012x over previous
import functools

import jax
import jax.numpy as jnp
from jax import lax
from jax.experimental import pallas as pl
from jax.experimental.pallas import tpu as pltpu
from jax.experimental.pallas import tpu_sc as plsc

_N_WORKERS = 32


def _gather_kernel(n_tokens, dim, chunk, nbuf, dist):
    per_w = n_tokens // _N_WORKERS
    n_chunks = per_w // chunk
    mesh = plsc.VectorSubcoreMesh(core_axis_name="c", subcore_axis_name="s")

    @functools.partial(
        pl.kernel,
        mesh=mesh,
        out_type=jax.ShapeDtypeStruct((n_tokens, dim), jnp.float32),
        scratch_types=(
            [
                pltpu.VMEM((per_w,), jnp.int32),
                pltpu.VMEM((nbuf, chunk, dim), jnp.float32),
            ]
            + [pltpu.SemaphoreType.DMA] * (2 * nbuf)
        ),
        compiler_params=pltpu.CompilerParams(use_tc_tiling_on_sc=False),
    )
    def k(idx_hbm, table_hbm, out_hbm, idx_v, rows_v, *sems):
        gsems = sems[:nbuf]
        wsems = sems[nbuf:]
        wid = lax.axis_index("s") * 2 + lax.axis_index("c")
        base = wid * per_w

        pltpu.sync_copy(idx_hbm.at[pl.ds(base, per_w)], idx_v)

        pending_g = {}
        pending_w = {}

        def start_gather(j):
            b = j % nbuf
            pending_g[b] = pltpu.async_copy(
                table_hbm.at[idx_v.at[pl.ds(j * chunk, chunk)]],
                rows_v.at[b],
                gsems[b],
            )

        for j in range(min(dist, n_chunks)):
            start_gather(j)
        for i in range(n_chunks):
            b = i % nbuf
            pending_g.pop(b).wait()
            pending_w[b] = pltpu.async_copy(
                rows_v.at[b], out_hbm.at[pl.ds(base + i * chunk, chunk)], wsems[b]
            )
            j = i + dist
            if j < n_chunks:
                bj = j % nbuf
                if bj in pending_w:
                    pending_w.pop(bj).wait()
                start_gather(j)
        for w in pending_w.values():
            w.wait()

    return k


def kernel(token_ids, weight):
    b, s = token_ids.shape
    v, d = weight.shape
    n = b * s
    chunk = 512
    assert (n // _N_WORKERS) % chunk == 0
    flat = token_ids.reshape(n).astype(jnp.int32)
    out = _gather_kernel(n, d, chunk, nbuf=3, dist=2)(flat, weight)
    return out.reshape(b, s, d)
